# baseline (speedup 1.0000x reference)
import jax
import jax.numpy as jnp
from jax.experimental import pallas as pl
from jax.experimental.pallas import tpu as pltpu

_TILE = 38168


def _matmul_kernel(x_ref, w_ref, b_ref, o_ref):
    o_ref[...] = (
        jnp.dot(x_ref[...], w_ref[...], preferred_element_type=jnp.float32)
        + b_ref[...]
    )


def kernel(input, kernel, bias):
    n, in_ch = input.shape
    out_ch = kernel.shape[1]
    grid = (pl.cdiv(n, _TILE),)
    return pl.pallas_call(
        _matmul_kernel,
        grid=grid,
        in_specs=[
            pl.BlockSpec((_TILE, in_ch), lambda i: (i, 0)),
            pl.BlockSpec((in_ch, out_ch), lambda i: (0, 0)),
            pl.BlockSpec((1, out_ch), lambda i: (0, 0)),
        ],
        out_specs=pl.BlockSpec(
            (_TILE, out_ch),
            lambda i: (i, 0),
            pipeline_mode=pl.Buffered(buffer_count=1),
        ),
        out_shape=jax.ShapeDtypeStruct((n, out_ch), jnp.float32),
        compiler_params=pltpu.CompilerParams(
            dimension_semantics=("arbitrary",),
        ),
    )(input, kernel, bias)


# speedup vs baseline: 1.3153x; 1.3153x over previous
import jax
import jax.numpy as jnp
from jax.experimental import pallas as pl
from jax.experimental.pallas import tpu as pltpu

_TILE = 28400


def _matmul_kernel(x_ref, w_ref, b_ref, o_ref):
    o_ref[...] = (
        jnp.dot(x_ref[...], w_ref[...], preferred_element_type=jnp.float32)
        + b_ref[...]
    )


def kernel(input, kernel, bias):
    n, in_ch = input.shape
    out_ch = kernel.shape[1]
    grid = (pl.cdiv(n, _TILE),)
    return pl.pallas_call(
        _matmul_kernel,
        grid=grid,
        in_specs=[
            pl.BlockSpec((_TILE, in_ch), lambda i: (i, 0)),
            pl.BlockSpec((in_ch, out_ch), lambda i: (0, 0)),
            pl.BlockSpec((1, out_ch), lambda i: (0, 0)),
        ],
        out_specs=pl.BlockSpec((_TILE, out_ch), lambda i: (i, 0)),
        out_shape=jax.ShapeDtypeStruct((n, out_ch), jnp.float32),
        compiler_params=pltpu.CompilerParams(
            dimension_semantics=("arbitrary",),
        ),
    )(input, kernel, bias)
